# speedup vs baseline: 7.3198x; 7.3198x over previous
import functools

import jax
import jax.numpy as jnp
from jax import lax
from jax.experimental import pallas as pl
from jax.experimental.pallas import tpu as pltpu
from jax.experimental.pallas import tpu_sc as plsc

N = 10000
E = 320000
D_IN = 128
D_OUT = 128

NC = 2
NS = 16
NW = NC * NS
NP = 10240
CH = 128
EPW = 10240
NCHUNK = EPW // CH
E_PAD = NW * EPW


def _tc_matmul(xp, W, normp):
    BM = 1280

    def body(x_ref, w_ref, n_ref, o_ref):
        o_ref[...] = jnp.dot(x_ref[...], w_ref[...],
                             preferred_element_type=jnp.float32) * n_ref[...]

    return pl.pallas_call(
        body,
        grid=(NP // BM,),
        in_specs=[
            pl.BlockSpec((BM, D_IN), lambda i: (i, 0)),
            pl.BlockSpec((D_IN, D_OUT), lambda i: (0, 0)),
            pl.BlockSpec((BM, 1), lambda i: (i, 0)),
        ],
        out_specs=pl.BlockSpec((BM, D_OUT), lambda i: (i, 0)),
        out_shape=jax.ShapeDtypeStruct((NP, D_OUT), jnp.float32),
    )(xp, W, normp)


def _sc_scatter(h2, srcw, dstw, zeros):
    mesh = plsc.VectorSubcoreMesh(core_axis_name="c", subcore_axis_name="s",
                                  num_cores=NC, num_subcores=NS)

    @functools.partial(
        pl.kernel,
        out_type=jax.ShapeDtypeStruct((NC, NP, D_OUT), jnp.float32),
        mesh=mesh,
        scratch_types=[
            pltpu.VMEM((NCHUNK, CH), jnp.int32),
            pltpu.VMEM((NCHUNK, CH), jnp.int32),
            pltpu.VMEM((CH, D_OUT), jnp.float32),
            pltpu.VMEM_SHARED((NP, D_OUT), jnp.float32),
            pltpu.SemaphoreType.DMA,
        ],
    )
    def k(h2_hbm, src_hbm, dst_hbm, z_hbm, part_hbm, src_v, dst_v, rows_v,
          acc, sem):
        cid = lax.axis_index("c")
        sid = lax.axis_index("s")
        wid = sid * NC + cid
        rpt = NP // NS
        r0 = sid * rpt
        pltpu.sync_copy(z_hbm.at[pl.ds(r0, rpt)], acc.at[pl.ds(r0, rpt)])
        pltpu.sync_copy(src_hbm.at[wid], src_v)
        pltpu.sync_copy(dst_hbm.at[wid], dst_v)
        plsc.subcore_barrier()

        def body(j, carry):
            pltpu.async_copy(h2_hbm.at[src_v.at[j]], rows_v, sem).wait()
            pltpu.sync_copy(rows_v, acc.at[dst_v.at[j]], add=True)
            return carry

        lax.fori_loop(0, NCHUNK, body, 0)
        plsc.subcore_barrier()
        pltpu.sync_copy(acc.at[pl.ds(r0, rpt)],
                        part_hbm.at[cid, pl.ds(r0, rpt)])

    return k(h2, srcw, dstw, zeros)


def _tc_combine(parts, norm, b2):
    BM = 1000

    def body(p_ref, n_ref, b_ref, o_ref):
        o_ref[...] = (p_ref[0] + p_ref[1]) * n_ref[...] + b_ref[...]

    return pl.pallas_call(
        body,
        grid=(N // BM,),
        in_specs=[
            pl.BlockSpec((NC, BM, D_OUT), lambda i: (0, i, 0)),
            pl.BlockSpec((BM, 1), lambda i: (i, 0)),
            pl.BlockSpec((1, D_OUT), lambda i: (0, 0)),
        ],
        out_specs=pl.BlockSpec((BM, D_OUT), lambda i: (i, 0)),
        out_shape=jax.ShapeDtypeStruct((N, D_OUT), jnp.float32),
    )(parts, norm, b2)


def kernel(x, edge_index, norm, W, b):
    src = edge_index[0]
    dst = edge_index[1]
    pad = jnp.full((E_PAD - E,), N, dtype=jnp.int32)
    srcw = jnp.concatenate([src, pad]).reshape(NW, NCHUNK, CH)
    dstw = jnp.concatenate([dst, pad]).reshape(NW, NCHUNK, CH)
    xp = jnp.pad(x, ((0, NP - N), (0, 0)))
    normp = jnp.pad(norm, ((0, NP - N), (0, 0)))
    h2 = _tc_matmul(xp, W, normp)
    zeros = jnp.zeros((NP, D_OUT), jnp.float32)
    parts = _sc_scatter(h2, srcw, dstw, zeros)
    return _tc_combine(parts, norm, b.reshape(1, D_OUT))


# speedup vs baseline: 17.1720x; 2.3460x over previous
import functools

import jax
import jax.numpy as jnp
from jax import lax
from jax.experimental import pallas as pl
from jax.experimental.pallas import tpu as pltpu
from jax.experimental.pallas import tpu_sc as plsc

N = 10000
E = 320000
D_IN = 128
D_OUT = 128

NC = 2
NS = 16
NW = NC * NS
NT = 10240
CH = 120
NCHUNK = 84
EPW = NCHUNK * CH
E_PAD = NW * EPW
NSLOT = 3
KR = 6
GA = NSLOT - 1
IA = KR - 1


def _tc_matmul(xp, W, normp):
    BM = 1280

    def body(x_ref, w_ref, n_ref, o_ref):
        o_ref[...] = jnp.dot(x_ref[...], w_ref[...],
                             preferred_element_type=jnp.float32) * n_ref[...]

    return pl.pallas_call(
        body,
        grid=(NT // BM,),
        in_specs=[
            pl.BlockSpec((BM, D_IN), lambda i: (i, 0)),
            pl.BlockSpec((D_IN, D_OUT), lambda i: (0, 0)),
            pl.BlockSpec((BM, 1), lambda i: (i, 0)),
        ],
        out_specs=pl.BlockSpec((BM, D_OUT), lambda i: (i, 0)),
        out_shape=jax.ShapeDtypeStruct((NT, D_OUT), jnp.float32),
    )(xp, W, normp)


def _sc_scatter(h2, srcw, dstw, zeros):
    mesh = plsc.VectorSubcoreMesh(core_axis_name="c", subcore_axis_name="s",
                                  num_cores=NC, num_subcores=NS)

    @functools.partial(
        pl.kernel,
        out_type=jax.ShapeDtypeStruct((NC, N, D_OUT), jnp.float32),
        mesh=mesh,
        scratch_types=[
            pltpu.VMEM((KR, CH), jnp.int32),
            pltpu.VMEM((KR, CH), jnp.int32),
            pltpu.VMEM((NSLOT, CH, D_OUT), jnp.float32),
            pltpu.VMEM_SHARED((N, D_OUT), jnp.float32),
            [pltpu.SemaphoreType.DMA] * NSLOT,
            [pltpu.SemaphoreType.DMA] * KR,
        ],
    )
    def k(h2_hbm, src_hbm, dst_hbm, z_hbm, part_hbm, src_v, dst_v, rows_v,
          acc, gsems, isems):
        cid = lax.axis_index("c")
        sid = lax.axis_index("s")
        wid = sid * NC + cid
        RPT = 632
        LAST = N - (NS - 1) * RPT
        r0 = sid * RPT

        def each_tile_rows(fn):
            @pl.when(sid < NS - 1)
            def _():
                fn(r0, RPT)

            @pl.when(sid == NS - 1)
            def _():
                fn((NS - 1) * RPT, LAST)

        each_tile_rows(lambda o, n: pltpu.sync_copy(
            z_hbm.at[pl.ds(o, n)], acc.at[pl.ds(o, n)]))
        plsc.subcore_barrier()

        def start_idx(j, slot):
            pltpu.async_copy(src_hbm.at[wid, j], src_v.at[slot], isems[slot])
            pltpu.async_copy(dst_hbm.at[wid, j], dst_v.at[slot], isems[slot])

        def wait_idx(j, slot):
            pltpu.make_async_copy(src_hbm.at[wid, j], src_v.at[slot],
                                  isems[slot]).wait()
            pltpu.make_async_copy(dst_hbm.at[wid, j], dst_v.at[slot],
                                  isems[slot]).wait()

        def start_gather(rslot, islot):
            pltpu.async_copy(h2_hbm.at[src_v.at[islot]], rows_v.at[rslot],
                             gsems[rslot])

        def wait_gather(rslot, islot):
            pltpu.make_async_copy(h2_hbm.at[src_v.at[islot]],
                                  rows_v.at[rslot], gsems[rslot]).wait()

        for t in range(IA):
            start_idx(t, t)
        for t in range(GA):
            wait_idx(t, t)
            start_gather(t, t)

        def body(g, carry):
            for b in range(KR):
                j = g * KR + b
                rb = b % NSLOT

                @pl.when(j + IA < NCHUNK)
                def _():
                    start_idx(j + IA, (b + IA) % KR)

                wait_gather(rb, b)

                @pl.when(j + GA < NCHUNK)
                def _():
                    wait_idx(j + GA, (b + GA) % KR)
                    start_gather((rb + GA) % NSLOT, (b + GA) % KR)

                pltpu.sync_copy(rows_v.at[rb], acc.at[dst_v.at[b]], add=True)
            return carry

        lax.fori_loop(0, NCHUNK // KR, body, 0)
        plsc.subcore_barrier()
        each_tile_rows(lambda o, n: pltpu.sync_copy(
            acc.at[pl.ds(o, n)], part_hbm.at[cid, pl.ds(o, n)]))

    return k(h2, srcw, dstw, zeros)


def _tc_combine(parts, norm, b2):
    BM = 1000

    def body(p_ref, n_ref, b_ref, o_ref):
        o_ref[...] = (p_ref[0] + p_ref[1]) * n_ref[...] + b_ref[...]

    return pl.pallas_call(
        body,
        grid=(N // BM,),
        in_specs=[
            pl.BlockSpec((NC, BM, D_OUT), lambda i: (0, i, 0)),
            pl.BlockSpec((BM, 1), lambda i: (i, 0)),
            pl.BlockSpec((1, D_OUT), lambda i: (0, 0)),
        ],
        out_specs=pl.BlockSpec((BM, D_OUT), lambda i: (i, 0)),
        out_shape=jax.ShapeDtypeStruct((N, D_OUT), jnp.float32),
    )(parts, norm, b2)


def kernel(x, edge_index, norm, W, b):
    src = edge_index[0]
    dst = edge_index[1]
    srcw = jnp.concatenate(
        [src, jnp.full((E_PAD - E,), N, dtype=jnp.int32)]).reshape(
            NW, NCHUNK, CH)
    dstw = jnp.concatenate(
        [dst, jnp.zeros((E_PAD - E,), dtype=jnp.int32)]).reshape(
            NW, NCHUNK, CH)
    xp = jnp.pad(x, ((0, NT - N), (0, 0)))
    normp = jnp.pad(norm, ((0, NT - N), (0, 0)))
    h2 = _tc_matmul(xp, W, normp)
    zeros = jnp.zeros((N, D_OUT), jnp.float32)
    parts = _sc_scatter(h2, srcw, dstw, zeros)
    return _tc_combine(parts, norm, b.reshape(1, D_OUT))
